# Initial kernel scaffold; baseline (speedup 1.0000x reference)
import jax
import jax.numpy as jnp
from jax.experimental import pallas as pl


def kernel(x, W1, b1, W2, b2, Wp, bp, c1w, c1b, c2w, c2b, Wl, bl, edge_index):
    raise NotImplementedError("write your pallas kernel here")


# speedup vs baseline: 1.2016x; 1.2016x over previous
import jax
import jax.numpy as jnp
from jax.experimental import pallas as pl
from jax.experimental.pallas import tpu as pltpu

N = 10000
E = 320000
K = 5


def _gcn_conv(x, src, dst, dis, W, b):
    n = x.shape[0]
    norm = dis[src] * dis[dst]
    h = x @ W
    out = jax.ops.segment_sum(h[src] * norm[:, None], dst, num_segments=n)
    out = out + h * (dis * dis)[:, None]
    return out + b


def _final_linear_kernel(x1_ref, x2_ref, wl_ref, bl_ref, o_ref):
    x1 = x1_ref[...]
    x2 = x2_ref[...]
    wl = wl_ref[...]
    o_ref[...] = (
        jnp.dot(x1, wl[:64], preferred_element_type=jnp.float32)
        + jnp.dot(x2, wl[64:], preferred_element_type=jnp.float32)
        + bl_ref[...]
    )


def kernel(x, W1, b1, W2, b2, Wp, bp, c1w, c1b, c2w, c2b, Wl, bl, edge_index):
    src, dst = edge_index[0], edge_index[1]
    ones = jnp.ones((E,), dtype=jnp.float32)
    deg = jax.ops.segment_sum(ones, dst, num_segments=N) + 1.0
    dis = jax.lax.rsqrt(deg)

    h = jax.nn.relu(_gcn_conv(x, src, dst, dis, W1, b1))
    x1 = _gcn_conv(h, src, dst, dis, W2, b2)

    g = x1 @ Wp + bp
    sort_idx = jnp.argsort(g[:, 0])
    inverse_idx = jnp.argsort(sort_idx)
    sorted_x = g[sort_idx] * x1[sort_idx]

    def conv(xs, w, b):
        out = jnp.zeros_like(xs)
        for k in range(K):
            sh = k - K // 2
            shifted = jnp.roll(xs, -sh, axis=0)
            if sh < 0:
                shifted = shifted.at[:(-sh)].set(0.0)
            elif sh > 0:
                shifted = shifted.at[-sh:].set(0.0)
            out = out + shifted @ w[:, :, k].T
        return out + b

    s = jax.nn.relu(conv(sorted_x, c1w, c1b))
    s = conv(s, c2w, c2b)
    x2 = s[inverse_idx]

    out = pl.pallas_call(
        _final_linear_kernel,
        out_shape=jax.ShapeDtypeStruct((N, 64), jnp.float32),
    )(x1, x2, Wl, bl.reshape(1, 64))
    return out


# speedup vs baseline: 10.5216x; 8.7562x over previous
import functools

import jax
import jax.numpy as jnp
from jax import lax
from jax.experimental import pallas as pl
from jax.experimental.pallas import tpu as pltpu
from jax.experimental.pallas import tpu_sc as plsc

N = 10000
E = 320000
D_IN = 128
HID = 64
C = 64
K = 5

NC = 2
NS = 16
NW = NC * NS

NPAD = 10240
EPW = 10240
EPAD = EPW * NW
BATCH = 128
KB = 8
SUPERS = EPW // (KB * BATCH)
ROWS_PT = NPAD // NS


@functools.cache
def _sc_mesh():
    return plsc.VectorSubcoreMesh(core_axis_name="c", subcore_axis_name="s",
                                  num_cores=NC, num_subcores=NS)


def _sc_deg_body(dst2d, deg_p, didx_v, ones_v, zeros_v, accum):
    c = lax.axis_index("c")
    s = lax.axis_index("s")
    w = c * NS + s
    for i in range(BATCH // 16):
        ones_v[pl.ds(i * 16, 16)] = jnp.ones((16,), jnp.float32)
    for i in range(ROWS_PT // 16):
        zeros_v[pl.ds(i * 16, 16)] = jnp.zeros((16,), jnp.float32)
    pltpu.sync_copy(zeros_v, accum.at[pl.ds(s * ROWS_PT, ROWS_PT)])
    plsc.subcore_barrier()

    row0 = w * (EPW // BATCH)

    def body(t, _):
        pltpu.sync_copy(dst2d.at[pl.ds(row0 + t * KB, KB)], didx_v)
        for b in range(KB):
            pltpu.sync_copy(ones_v, accum.at[didx_v.at[b]], add=True)
        return 0

    lax.fori_loop(0, SUPERS, body, 0)
    plsc.subcore_barrier()
    pltpu.sync_copy(accum.at[pl.ds(s * ROWS_PT, ROWS_PT)],
                    deg_p.at[c, 0, pl.ds(s * ROWS_PT, ROWS_PT)])


def _sc_deg(dst2d):
    return pl.kernel(
        _sc_deg_body,
        out_type=jax.ShapeDtypeStruct((NC, 1, NPAD), jnp.float32),
        mesh=_sc_mesh(),
        compiler_params=pltpu.CompilerParams(use_tc_tiling_on_sc=False),
        scratch_types=[
            pltpu.VMEM((KB, BATCH), jnp.int32),
            pltpu.VMEM((BATCH,), jnp.float32),
            pltpu.VMEM((ROWS_PT,), jnp.float32),
            pltpu.VMEM_SHARED((NPAD,), jnp.float32),
        ],
    )(dst2d)


def _sc_agg_body(hs, src2d, dst2d, out_p, sidx_v, didx_v, rows_v, accum,
                 gsem, ssem):
    c = lax.axis_index("c")
    s = lax.axis_index("s")
    w = c * NS + s
    pltpu.sync_copy(hs.at[pl.ds(s * ROWS_PT, ROWS_PT)],
                    accum.at[pl.ds(s * ROWS_PT, ROWS_PT)])
    plsc.subcore_barrier()

    row0 = w * (EPW // BATCH)

    def body(t, _):
        base = row0 + t * KB
        pltpu.sync_copy(src2d.at[pl.ds(base, KB)], sidx_v)
        pltpu.sync_copy(dst2d.at[pl.ds(base, KB)], didx_v)
        descs = [pltpu.async_copy(hs.at[sidx_v.at[b]], rows_v.at[b], gsem)
                 for b in range(KB)]
        for d in descs:
            d.wait()
        descs = [pltpu.async_copy(rows_v.at[b], accum.at[didx_v.at[b]], ssem,
                                  add=True) for b in range(KB)]
        for d in descs:
            d.wait()
        return 0

    lax.fori_loop(0, SUPERS, body, 0)
    plsc.subcore_barrier()
    pltpu.sync_copy(accum.at[pl.ds(s * ROWS_PT, ROWS_PT)],
                    out_p.at[c, pl.ds(s * ROWS_PT, ROWS_PT)])


def _sc_agg(hs_pad, src2d, dst2d):
    return pl.kernel(
        _sc_agg_body,
        out_type=jax.ShapeDtypeStruct((NC, NPAD, C), jnp.float32),
        mesh=_sc_mesh(),
        compiler_params=pltpu.CompilerParams(use_tc_tiling_on_sc=False),
        scratch_types=[
            pltpu.VMEM((KB, BATCH), jnp.int32),
            pltpu.VMEM((KB, BATCH), jnp.int32),
            pltpu.VMEM((KB, BATCH, C), jnp.float32),
            pltpu.VMEM_SHARED((NPAD, C), jnp.float32),
            pltpu.SemaphoreType.DMA,
            pltpu.SemaphoreType.DMA,
        ],
    )(hs_pad, src2d, dst2d)


def _sc_scat_body(y, op, rank3d, sy, sop, ridx_v, rows_v, rows2_v, sem):
    c = lax.axis_index("c")
    s = lax.axis_index("s")
    w = c * NS + s

    def do_batch(b):
        pltpu.sync_copy(rank3d.at[b, 0], ridx_v)
        pltpu.sync_copy(y.at[pl.ds(b * BATCH, BATCH)], rows_v)
        pltpu.sync_copy(op.at[pl.ds(b * BATCH, BATCH)], rows2_v)
        pltpu.async_copy(rows_v, sy.at[ridx_v], sem).wait()
        pltpu.async_copy(rows2_v, sop.at[ridx_v], sem).wait()

    do_batch(w * 2)
    do_batch(w * 2 + 1)

    @pl.when(w < NPAD // BATCH - 2 * NW)
    def _():
        do_batch(2 * NW + w)


def _sc_scat(y_pad, op_pad, rank3d):
    return pl.kernel(
        _sc_scat_body,
        out_type=(jax.ShapeDtypeStruct((NPAD, C), jnp.float32),
                  jax.ShapeDtypeStruct((NPAD, C), jnp.float32)),
        mesh=_sc_mesh(),
        compiler_params=pltpu.CompilerParams(use_tc_tiling_on_sc=False),
        scratch_types=[
            pltpu.VMEM((BATCH,), jnp.int32),
            pltpu.VMEM((BATCH, C), jnp.float32),
            pltpu.VMEM((BATCH, C), jnp.float32),
            pltpu.SemaphoreType.DMA,
        ],
    )(y_pad, op_pad, rank3d)


def _sc_gath_body(z, rank3d, out, ridx_v, rows_v, sem):
    c = lax.axis_index("c")
    s = lax.axis_index("s")
    w = c * NS + s

    def do_batch(b):
        pltpu.sync_copy(rank3d.at[b, 0], ridx_v)
        pltpu.async_copy(z.at[ridx_v], rows_v, sem).wait()
        pltpu.sync_copy(rows_v, out.at[pl.ds(b * BATCH, BATCH)])

    do_batch(w * 2)
    do_batch(w * 2 + 1)

    @pl.when(w < NPAD // BATCH - 2 * NW)
    def _():
        do_batch(2 * NW + w)


def _sc_gath(z, rank3d):
    return pl.kernel(
        _sc_gath_body,
        out_type=jax.ShapeDtypeStruct((NPAD, C), jnp.float32),
        mesh=_sc_mesh(),
        compiler_params=pltpu.CompilerParams(use_tc_tiling_on_sc=False),
        scratch_types=[
            pltpu.VMEM((BATCH,), jnp.int32),
            pltpu.VMEM((BATCH, C), jnp.float32),
            pltpu.SemaphoreType.DMA,
        ],
    )(z, rank3d)


def _tc1_body(x_ref, w1_ref, degt_ref, hs_ref, dis_ref):
    deg = degt_ref[:, 0:1] + degt_ref[:, 1:2] + 1.0
    dis = lax.rsqrt(deg)
    xw = jnp.dot(x_ref[...], w1_ref[...], preferred_element_type=jnp.float32)
    hs_ref[0:N, :] = xw * dis
    hs_ref[N:NPAD, :] = jnp.zeros((NPAD - N, HID), jnp.float32)
    dis_ref[...] = dis


def _tc1(x, W1, degt):
    return pl.pallas_call(
        _tc1_body,
        out_shape=(jax.ShapeDtypeStruct((NPAD, HID), jnp.float32),
                   jax.ShapeDtypeStruct((N, 1), jnp.float32)),
    )(x, W1, degt)


def _tc3_body(p_ref, hs1_ref, dis_ref, b1_ref, w2_ref, hs2_ref):
    agg = p_ref[0, 0:N, :] + p_ref[1, 0:N, :] - hs1_ref[0:N, :]
    h = jnp.maximum(agg * dis_ref[...] + b1_ref[...], 0.0)
    hw = jnp.dot(h, w2_ref[...], preferred_element_type=jnp.float32)
    hs2_ref[0:N, :] = hw * dis_ref[...]
    hs2_ref[N:NPAD, :] = jnp.zeros((NPAD - N, C), jnp.float32)


def _tc3(p, hs1, dis, b1, W2):
    return pl.pallas_call(
        _tc3_body,
        out_shape=jax.ShapeDtypeStruct((NPAD, C), jnp.float32),
    )(p, hs1, dis, b1, W2)


def _tc5_body(p_ref, hs2_ref, dis_ref, b2_ref, wp_ref, bp_ref, wlt_ref,
              bl_ref, y_ref, g_ref, op_ref):
    agg = p_ref[0, 0:N, :] + p_ref[1, 0:N, :] - hs2_ref[0:N, :]
    x1 = agg * dis_ref[...] + b2_ref[...]
    g = jnp.dot(x1, wp_ref[...], preferred_element_type=jnp.float32) + bp_ref[...]
    y_ref[0:N, :] = g * x1
    y_ref[N:NPAD, :] = jnp.zeros((NPAD - N, C), jnp.float32)
    g_ref[0:N, :] = g
    g_ref[N:NPAD, :] = jnp.full((NPAD - N, 1), jnp.inf, jnp.float32)
    op_ref[0:N, :] = (jnp.dot(x1, wlt_ref[...], preferred_element_type=jnp.float32)
                      + bl_ref[...])
    op_ref[N:NPAD, :] = jnp.zeros((NPAD - N, C), jnp.float32)


def _tc5(p, hs2, dis, b2, Wp, bp, Wl_top, bl):
    return pl.pallas_call(
        _tc5_body,
        out_shape=(jax.ShapeDtypeStruct((NPAD, C), jnp.float32),
                   jax.ShapeDtypeStruct((NPAD, 1), jnp.float32),
                   jax.ShapeDtypeStruct((NPAD, C), jnp.float32)),
    )(p, hs2, dis, b2, Wp, bp, Wl_top, bl)


def _key(v):
    b = lax.bitcast_convert_type(v, jnp.uint32)
    u = b ^ jnp.where(b >= jnp.uint32(0x80000000), jnp.uint32(0xFFFFFFFF),
                      jnp.uint32(0x80000000))
    return (u ^ jnp.uint32(0x80000000)).astype(jnp.int32)


RANK_BI = 256
RANK_BJ = 2048


def _rank_body(gcol_ref, grow_ref, rank_ref):
    i0 = pl.program_id(0) * RANK_BI
    ki = _key(gcol_ref[...])
    ii = lax.broadcasted_iota(jnp.int32, (RANK_BI, 1), 0) + i0
    acc = jnp.zeros((RANK_BI, 1), jnp.int32)
    for cj in range(NPAD // RANK_BJ):
        kj = _key(grow_ref[0:1, cj * RANK_BJ:(cj + 1) * RANK_BJ])
        jj = lax.broadcasted_iota(jnp.int32, (1, RANK_BJ), 1) + cj * RANK_BJ
        p = (kj < ki) | ((kj == ki) & (jj < ii))
        acc = acc + jnp.sum(p.astype(jnp.int32), axis=1, keepdims=True)
    rank_ref[...] = acc


def _rank(gcol, grow):
    return pl.pallas_call(
        _rank_body,
        grid=(NPAD // RANK_BI,),
        in_specs=[
            pl.BlockSpec((RANK_BI, 1), lambda i: (i, 0)),
            pl.BlockSpec((1, NPAD), lambda i: (0, 0)),
        ],
        out_specs=pl.BlockSpec((RANK_BI, 1), lambda i: (i, 0)),
        out_shape=jax.ShapeDtypeStruct((NPAD, 1), jnp.int32),
    )(gcol, grow)


def _conv7_body(sy_ref, sop_ref, w1s_ref, b1c_ref, w2s_ref, b2c_ref,
                wlb_ref, z_ref):
    def conv(xin, ws_ref, brow):
        out = brow
        for k in range(K):
            sh = k - K // 2
            if sh < 0:
                shifted = jnp.concatenate(
                    [jnp.zeros((-sh, C), jnp.float32), xin[: NPAD + sh]], axis=0)
            elif sh > 0:
                shifted = jnp.concatenate(
                    [xin[sh:], jnp.zeros((sh, C), jnp.float32)], axis=0)
            else:
                shifted = xin
            wk = ws_ref[pl.ds(k * C, C), :]
            out = out + jnp.dot(shifted, wk, preferred_element_type=jnp.float32)
        return out

    sy = sy_ref[...]
    s1 = jnp.maximum(conv(sy, w1s_ref, b1c_ref[...]), 0.0)
    rowmask = (lax.broadcasted_iota(jnp.int32, (NPAD, 1), 0) < N)
    s1 = jnp.where(rowmask, s1, 0.0)
    s2 = conv(s1, w2s_ref, b2c_ref[...])
    z_ref[...] = (jnp.dot(s2, wlb_ref[...], preferred_element_type=jnp.float32)
                  + sop_ref[...])


def _conv7(sy, sop, w1s, b1c, w2s, b2c, Wl_bot):
    return pl.pallas_call(
        _conv7_body,
        out_shape=jax.ShapeDtypeStruct((NPAD, C), jnp.float32),
    )(sy, sop, w1s, b1c, w2s, b2c, Wl_bot)


def kernel(x, W1, b1, W2, b2, Wp, bp, c1w, c1b, c2w, c2b, Wl, bl, edge_index):
    src, dst = edge_index[0], edge_index[1]
    pad = EPAD - E
    src_pad = jnp.concatenate([src, jnp.zeros((pad,), jnp.int32)])
    dst_pad = jnp.concatenate([dst, jnp.full((pad,), N, jnp.int32)])
    src2d = src_pad.reshape(EPAD // BATCH, BATCH)
    dst2d = dst_pad.reshape(EPAD // BATCH, BATCH)

    deg_p = _sc_deg(dst2d)
    degt = jnp.concatenate([deg_p[0, 0, :N].reshape(N, 1),
                            deg_p[1, 0, :N].reshape(N, 1)], axis=1)

    hs1, dis = _tc1(x, W1, degt)
    p1 = _sc_agg(hs1, src2d, dst2d)

    hs2 = _tc3(p1, hs1, dis, b1.reshape(1, HID), W2)
    p2 = _sc_agg(hs2, src2d, dst2d)

    Wl_top, Wl_bot = Wl[:C], Wl[C:]
    y, g, op = _tc5(p2, hs2, dis, b2.reshape(1, C), Wp, bp.reshape(1, 1),
                    Wl_top, bl.reshape(1, C))

    grow = g.reshape(1, NPAD)
    rank_col = _rank(g, grow)
    rank3d = rank_col.reshape(NPAD // BATCH, 1, BATCH)

    sy, sop = _sc_scat(y, op, rank3d)

    w1s = jnp.concatenate([c1w[:, :, k].T for k in range(K)], axis=0)
    w2s = jnp.concatenate([c2w[:, :, k].T for k in range(K)], axis=0)
    z = _conv7(sy, sop, w1s, c1b.reshape(1, C), w2s, c2b.reshape(1, C), Wl_bot)

    out_pad = _sc_gath(z, rank3d)
    return out_pad[:N]


# speedup vs baseline: 10.8357x; 1.0299x over previous
import functools

import jax
import jax.numpy as jnp
from jax import lax
from jax.experimental import pallas as pl
from jax.experimental.pallas import tpu as pltpu
from jax.experimental.pallas import tpu_sc as plsc

N = 10000
E = 320000
D_IN = 128
HID = 64
C = 64
K = 5

NC = 2
NS = 16
NW = NC * NS

NPAD = 10240
EPW = 10240
EPAD = EPW * NW
BATCH = 128
KB = 4
SUPERS = EPW // (KB * BATCH)
ROWS_PT = NPAD // NS


@functools.cache
def _sc_mesh():
    return plsc.VectorSubcoreMesh(core_axis_name="c", subcore_axis_name="s",
                                  num_cores=NC, num_subcores=NS)


def _sc_deg_body(dst2d, deg_p, didx_v, ones_v, zeros_v, accum):
    c = lax.axis_index("c")
    s = lax.axis_index("s")
    w = c * NS + s
    for i in range(BATCH // 16):
        ones_v[pl.ds(i * 16, 16)] = jnp.ones((16,), jnp.float32)
    for i in range(ROWS_PT // 16):
        zeros_v[pl.ds(i * 16, 16)] = jnp.zeros((16,), jnp.float32)
    pltpu.sync_copy(zeros_v, accum.at[pl.ds(s * ROWS_PT, ROWS_PT)])
    plsc.subcore_barrier()

    row0 = w * (EPW // BATCH)

    def body(t, _):
        pltpu.sync_copy(dst2d.at[pl.ds(row0 + t * 8, 8)], didx_v)
        for b in range(8):
            pltpu.sync_copy(ones_v, accum.at[didx_v.at[b]], add=True)
        return 0

    lax.fori_loop(0, EPW // (8 * BATCH), body, 0)
    plsc.subcore_barrier()
    pltpu.sync_copy(accum.at[pl.ds(s * ROWS_PT, ROWS_PT)],
                    deg_p.at[c, 0, pl.ds(s * ROWS_PT, ROWS_PT)])


def _sc_deg(dst2d):
    return pl.kernel(
        _sc_deg_body,
        out_type=jax.ShapeDtypeStruct((NC, 1, NPAD), jnp.float32),
        mesh=_sc_mesh(),
        compiler_params=pltpu.CompilerParams(use_tc_tiling_on_sc=False),
        scratch_types=[
            pltpu.VMEM((8, BATCH), jnp.int32),
            pltpu.VMEM((BATCH,), jnp.float32),
            pltpu.VMEM((ROWS_PT,), jnp.float32),
            pltpu.VMEM_SHARED((NPAD,), jnp.float32),
        ],
    )(dst2d)


def _sc_agg_body(hs, src2d, dst2d, out_p, sidx_v, didx_v, rows0_v, rows1_v,
                 accum, gsem, ssem):
    c = lax.axis_index("c")
    s = lax.axis_index("s")
    w = c * NS + s
    pltpu.sync_copy(hs.at[pl.ds(s * ROWS_PT, ROWS_PT)],
                    accum.at[pl.ds(s * ROWS_PT, ROWS_PT)])
    plsc.subcore_barrier()

    row0 = w * (EPW // BATCH)
    rows = (rows0_v, rows1_v)

    def load_pair(p):
        h = (p % 2) * 2 * KB
        pltpu.sync_copy(src2d.at[pl.ds(row0 + p * 2 * KB, 2 * KB)],
                        sidx_v.at[pl.ds(h, 2 * KB)])
        pltpu.sync_copy(dst2d.at[pl.ds(row0 + p * 2 * KB, 2 * KB)],
                        didx_v.at[pl.ds(h, 2 * KB)])

    def idx_row(t, b):
        return ((t // 2) % 2) * 2 * KB + (t % 2) * KB + b

    def fire_gather(t):
        return [pltpu.async_copy(hs.at[sidx_v.at[idx_row(t, b)]],
                                 rows[t % 2].at[b], gsem) for b in range(KB)]

    def fire_scatter(t):
        return [pltpu.async_copy(rows[t % 2].at[b],
                                 accum.at[didx_v.at[idx_row(t, b)]], ssem,
                                 add=True) for b in range(KB)]

    load_pair(0)
    g_in = fire_gather(0)
    s_in = None
    for t in range(SUPERS):
        for d in g_in:
            d.wait()
        if s_in is not None:
            for d in s_in:
                d.wait()
        if t + 1 < SUPERS:
            if t % 2 == 1:
                load_pair((t + 1) // 2)
            g_in = fire_gather(t + 1)
        s_in = fire_scatter(t)
    for d in s_in:
        d.wait()

    plsc.subcore_barrier()
    pltpu.sync_copy(accum.at[pl.ds(s * ROWS_PT, ROWS_PT)],
                    out_p.at[c, pl.ds(s * ROWS_PT, ROWS_PT)])


def _sc_agg(hs_pad, src2d, dst2d):
    return pl.kernel(
        _sc_agg_body,
        out_type=jax.ShapeDtypeStruct((NC, NPAD, C), jnp.float32),
        mesh=_sc_mesh(),
        compiler_params=pltpu.CompilerParams(use_tc_tiling_on_sc=False),
        scratch_types=[
            pltpu.VMEM((4 * KB, BATCH), jnp.int32),
            pltpu.VMEM((4 * KB, BATCH), jnp.int32),
            pltpu.VMEM((KB, BATCH, C), jnp.float32),
            pltpu.VMEM((KB, BATCH, C), jnp.float32),
            pltpu.VMEM_SHARED((NPAD, C), jnp.float32),
            pltpu.SemaphoreType.DMA,
            pltpu.SemaphoreType.DMA,
        ],
    )(hs_pad, src2d, dst2d)


def _sc_scat_body(y, op, rank3d, sy, sop, ridx_v, rows_v, rows2_v, sem):
    c = lax.axis_index("c")
    s = lax.axis_index("s")
    w = c * NS + s

    def do_batch(b):
        pltpu.sync_copy(rank3d.at[b, 0], ridx_v)
        pltpu.sync_copy(y.at[pl.ds(b * BATCH, BATCH)], rows_v)
        pltpu.sync_copy(op.at[pl.ds(b * BATCH, BATCH)], rows2_v)
        pltpu.async_copy(rows_v, sy.at[ridx_v], sem).wait()
        pltpu.async_copy(rows2_v, sop.at[ridx_v], sem).wait()

    do_batch(w * 2)
    do_batch(w * 2 + 1)

    @pl.when(w < NPAD // BATCH - 2 * NW)
    def _():
        do_batch(2 * NW + w)


def _sc_scat(y_pad, op_pad, rank3d):
    return pl.kernel(
        _sc_scat_body,
        out_type=(jax.ShapeDtypeStruct((NPAD, C), jnp.float32),
                  jax.ShapeDtypeStruct((NPAD, C), jnp.float32)),
        mesh=_sc_mesh(),
        compiler_params=pltpu.CompilerParams(use_tc_tiling_on_sc=False),
        scratch_types=[
            pltpu.VMEM((BATCH,), jnp.int32),
            pltpu.VMEM((BATCH, C), jnp.float32),
            pltpu.VMEM((BATCH, C), jnp.float32),
            pltpu.SemaphoreType.DMA,
        ],
    )(y_pad, op_pad, rank3d)


def _sc_gath_body(z, rank3d, out, ridx_v, rows_v, sem):
    c = lax.axis_index("c")
    s = lax.axis_index("s")
    w = c * NS + s

    def do_batch(b):
        pltpu.sync_copy(rank3d.at[b, 0], ridx_v)
        pltpu.async_copy(z.at[ridx_v], rows_v, sem).wait()
        pltpu.sync_copy(rows_v, out.at[pl.ds(b * BATCH, BATCH)])

    do_batch(w * 2)
    do_batch(w * 2 + 1)

    @pl.when(w < NPAD // BATCH - 2 * NW)
    def _():
        do_batch(2 * NW + w)


def _sc_gath(z, rank3d):
    return pl.kernel(
        _sc_gath_body,
        out_type=jax.ShapeDtypeStruct((NPAD, C), jnp.float32),
        mesh=_sc_mesh(),
        compiler_params=pltpu.CompilerParams(use_tc_tiling_on_sc=False),
        scratch_types=[
            pltpu.VMEM((BATCH,), jnp.int32),
            pltpu.VMEM((BATCH, C), jnp.float32),
            pltpu.SemaphoreType.DMA,
        ],
    )(z, rank3d)


def _tc1_body(x_ref, w1_ref, degt_ref, hs_ref, dis_ref):
    deg = degt_ref[:, 0:1] + degt_ref[:, 1:2] + 1.0
    dis = lax.rsqrt(deg)
    xw = jnp.dot(x_ref[...], w1_ref[...], preferred_element_type=jnp.float32)
    hs_ref[0:N, :] = xw * dis
    hs_ref[N:NPAD, :] = jnp.zeros((NPAD - N, HID), jnp.float32)
    dis_ref[...] = dis


def _tc1(x, W1, degt):
    return pl.pallas_call(
        _tc1_body,
        out_shape=(jax.ShapeDtypeStruct((NPAD, HID), jnp.float32),
                   jax.ShapeDtypeStruct((N, 1), jnp.float32)),
    )(x, W1, degt)


def _tc3_body(p_ref, hs1_ref, dis_ref, b1_ref, w2_ref, hs2_ref):
    agg = p_ref[0, 0:N, :] + p_ref[1, 0:N, :] - hs1_ref[0:N, :]
    h = jnp.maximum(agg * dis_ref[...] + b1_ref[...], 0.0)
    hw = jnp.dot(h, w2_ref[...], preferred_element_type=jnp.float32)
    hs2_ref[0:N, :] = hw * dis_ref[...]
    hs2_ref[N:NPAD, :] = jnp.zeros((NPAD - N, C), jnp.float32)


def _tc3(p, hs1, dis, b1, W2):
    return pl.pallas_call(
        _tc3_body,
        out_shape=jax.ShapeDtypeStruct((NPAD, C), jnp.float32),
    )(p, hs1, dis, b1, W2)


def _tc5_body(p_ref, hs2_ref, dis_ref, b2_ref, wp_ref, bp_ref, wlt_ref,
              bl_ref, y_ref, g_ref, op_ref):
    agg = p_ref[0, 0:N, :] + p_ref[1, 0:N, :] - hs2_ref[0:N, :]
    x1 = agg * dis_ref[...] + b2_ref[...]
    g = jnp.dot(x1, wp_ref[...], preferred_element_type=jnp.float32) + bp_ref[...]
    y_ref[0:N, :] = g * x1
    y_ref[N:NPAD, :] = jnp.zeros((NPAD - N, C), jnp.float32)
    g_ref[0:N, :] = g
    g_ref[N:NPAD, :] = jnp.full((NPAD - N, 1), jnp.inf, jnp.float32)
    op_ref[0:N, :] = (jnp.dot(x1, wlt_ref[...], preferred_element_type=jnp.float32)
                      + bl_ref[...])
    op_ref[N:NPAD, :] = jnp.zeros((NPAD - N, C), jnp.float32)


def _tc5(p, hs2, dis, b2, Wp, bp, Wl_top, bl):
    return pl.pallas_call(
        _tc5_body,
        out_shape=(jax.ShapeDtypeStruct((NPAD, C), jnp.float32),
                   jax.ShapeDtypeStruct((NPAD, 1), jnp.float32),
                   jax.ShapeDtypeStruct((NPAD, C), jnp.float32)),
    )(p, hs2, dis, b2, Wp, bp, Wl_top, bl)


def _key(v):
    b = lax.bitcast_convert_type(v, jnp.uint32)
    u = b ^ jnp.where(b >= jnp.uint32(0x80000000), jnp.uint32(0xFFFFFFFF),
                      jnp.uint32(0x80000000))
    return (u ^ jnp.uint32(0x80000000)).astype(jnp.int32)


RANK_BI = 256
RANK_BJ = 2048


def _rank_body(gcol_ref, grow_ref, rank_ref):
    i0 = pl.program_id(0) * RANK_BI
    ki = _key(gcol_ref[...])
    ii = lax.broadcasted_iota(jnp.int32, (RANK_BI, 1), 0) + i0
    acc = jnp.zeros((RANK_BI, 1), jnp.int32)
    for cj in range(NPAD // RANK_BJ):
        kj = _key(grow_ref[0:1, cj * RANK_BJ:(cj + 1) * RANK_BJ])
        jj = lax.broadcasted_iota(jnp.int32, (1, RANK_BJ), 1) + cj * RANK_BJ
        p = (kj < ki) | ((kj == ki) & (jj < ii))
        acc = acc + jnp.sum(p.astype(jnp.int32), axis=1, keepdims=True)
    rank_ref[...] = acc


def _rank(gcol, grow):
    return pl.pallas_call(
        _rank_body,
        grid=(NPAD // RANK_BI,),
        in_specs=[
            pl.BlockSpec((RANK_BI, 1), lambda i: (i, 0)),
            pl.BlockSpec((1, NPAD), lambda i: (0, 0)),
        ],
        out_specs=pl.BlockSpec((RANK_BI, 1), lambda i: (i, 0)),
        out_shape=jax.ShapeDtypeStruct((NPAD, 1), jnp.int32),
    )(gcol, grow)


def _conv7_body(sy_ref, sop_ref, w1s_ref, b1c_ref, w2s_ref, b2c_ref,
                wlb_ref, z_ref):
    def conv(xin, ws_ref, brow):
        out = brow
        for k in range(K):
            sh = k - K // 2
            if sh < 0:
                shifted = jnp.concatenate(
                    [jnp.zeros((-sh, C), jnp.float32), xin[: NPAD + sh]], axis=0)
            elif sh > 0:
                shifted = jnp.concatenate(
                    [xin[sh:], jnp.zeros((sh, C), jnp.float32)], axis=0)
            else:
                shifted = xin
            wk = ws_ref[pl.ds(k * C, C), :]
            out = out + jnp.dot(shifted, wk, preferred_element_type=jnp.float32)
        return out

    sy = sy_ref[...]
    s1 = jnp.maximum(conv(sy, w1s_ref, b1c_ref[...]), 0.0)
    rowmask = (lax.broadcasted_iota(jnp.int32, (NPAD, 1), 0) < N)
    s1 = jnp.where(rowmask, s1, 0.0)
    s2 = conv(s1, w2s_ref, b2c_ref[...])
    z_ref[...] = (jnp.dot(s2, wlb_ref[...], preferred_element_type=jnp.float32)
                  + sop_ref[...])


def _conv7(sy, sop, w1s, b1c, w2s, b2c, Wl_bot):
    return pl.pallas_call(
        _conv7_body,
        out_shape=jax.ShapeDtypeStruct((NPAD, C), jnp.float32),
    )(sy, sop, w1s, b1c, w2s, b2c, Wl_bot)


def kernel(x, W1, b1, W2, b2, Wp, bp, c1w, c1b, c2w, c2b, Wl, bl, edge_index):
    src, dst = edge_index[0], edge_index[1]
    pad = EPAD - E
    src_pad = jnp.concatenate([src, jnp.zeros((pad,), jnp.int32)])
    dst_pad = jnp.concatenate([dst, jnp.full((pad,), N, jnp.int32)])
    src2d = src_pad.reshape(EPAD // BATCH, BATCH)
    dst2d = dst_pad.reshape(EPAD // BATCH, BATCH)

    deg_p = _sc_deg(dst2d)
    degt = jnp.concatenate([deg_p[0, 0, :N].reshape(N, 1),
                            deg_p[1, 0, :N].reshape(N, 1)], axis=1)

    hs1, dis = _tc1(x, W1, degt)
    p1 = _sc_agg(hs1, src2d, dst2d)

    hs2 = _tc3(p1, hs1, dis, b1.reshape(1, HID), W2)
    p2 = _sc_agg(hs2, src2d, dst2d)

    Wl_top, Wl_bot = Wl[:C], Wl[C:]
    y, g, op = _tc5(p2, hs2, dis, b2.reshape(1, C), Wp, bp.reshape(1, 1),
                    Wl_top, bl.reshape(1, C))

    grow = g.reshape(1, NPAD)
    rank_col = _rank(g, grow)
    rank3d = rank_col.reshape(NPAD // BATCH, 1, BATCH)

    sy, sop = _sc_scat(y, op, rank3d)

    w1s = jnp.concatenate([c1w[:, :, k].T for k in range(K)], axis=0)
    w2s = jnp.concatenate([c2w[:, :, k].T for k in range(K)], axis=0)
    z = _conv7(sy, sop, w1s, c1b.reshape(1, C), w2s, c2b.reshape(1, C), Wl_bot)

    out_pad = _sc_gath(z, rank3d)
    return out_pad[:N]


# speedup vs baseline: 10.8587x; 1.0021x over previous
import functools

import jax
import jax.numpy as jnp
from jax import lax
from jax.experimental import pallas as pl
from jax.experimental.pallas import tpu as pltpu
from jax.experimental.pallas import tpu_sc as plsc

N = 10000
E = 320000
D_IN = 128
HID = 64
C = 64
K = 5

NC = 2
NS = 16
NW = NC * NS

NPAD = 10240
EPW = 10240
EPAD = EPW * NW
BATCH = 128
KB = 4
SUPERS = EPW // (KB * BATCH)
ROWS_PT = NPAD // NS


@functools.cache
def _sc_mesh():
    return plsc.VectorSubcoreMesh(core_axis_name="c", subcore_axis_name="s",
                                  num_cores=NC, num_subcores=NS)


def _sc_deg_body(dst2d, deg_p, didx_v, ones_v, zeros_v, accum):
    c = lax.axis_index("c")
    s = lax.axis_index("s")
    w = c * NS + s
    for i in range(BATCH // 16):
        ones_v[pl.ds(i * 16, 16)] = jnp.ones((16,), jnp.float32)
    for i in range(ROWS_PT // 16):
        zeros_v[pl.ds(i * 16, 16)] = jnp.zeros((16,), jnp.float32)
    pltpu.sync_copy(zeros_v, accum.at[pl.ds(s * ROWS_PT, ROWS_PT)])
    plsc.subcore_barrier()

    row0 = w * (EPW // BATCH)

    def body(t, _):
        pltpu.sync_copy(dst2d.at[pl.ds(row0 + t * 8, 8)], didx_v)
        for b in range(8):
            pltpu.sync_copy(ones_v, accum.at[didx_v.at[b]], add=True)
        return 0

    lax.fori_loop(0, EPW // (8 * BATCH), body, 0)
    plsc.subcore_barrier()
    pltpu.sync_copy(accum.at[pl.ds(s * ROWS_PT, ROWS_PT)],
                    deg_p.at[c, 0, pl.ds(s * ROWS_PT, ROWS_PT)])


def _sc_deg(dst2d):
    return pl.kernel(
        _sc_deg_body,
        out_type=jax.ShapeDtypeStruct((NC, 1, NPAD), jnp.float32),
        mesh=_sc_mesh(),
        compiler_params=pltpu.CompilerParams(use_tc_tiling_on_sc=False),
        scratch_types=[
            pltpu.VMEM((8, BATCH), jnp.int32),
            pltpu.VMEM((BATCH,), jnp.float32),
            pltpu.VMEM((ROWS_PT,), jnp.float32),
            pltpu.VMEM_SHARED((NPAD,), jnp.float32),
        ],
    )(dst2d)


def _sc_agg_body(hs, src1d, dst2d, out_p, sidx_v, didx_v, rows0_v, rows1_v,
                 accum, gsem, ssem):
    c = lax.axis_index("c")
    s = lax.axis_index("s")
    w = c * NS + s
    pltpu.sync_copy(hs.at[pl.ds(s * ROWS_PT, ROWS_PT)],
                    accum.at[pl.ds(s * ROWS_PT, ROWS_PT)])
    plsc.subcore_barrier()

    e0 = w * EPW
    row0 = w * (EPW // BATCH)
    rows = (rows0_v, rows1_v)
    SUP = KB * BATCH

    def load_pair(p):
        h = (p % 2) * 2 * KB
        pltpu.sync_copy(src1d.at[pl.ds(e0 + p * 2 * SUP, 2 * SUP)],
                        sidx_v.at[pl.ds(h * BATCH, 2 * SUP)])
        pltpu.sync_copy(dst2d.at[pl.ds(row0 + p * 2 * KB, 2 * KB)],
                        didx_v.at[pl.ds(h, 2 * KB)])

    def idx_row(t, b):
        return ((t // 2) % 2) * 2 * KB + (t % 2) * KB + b

    def fire_gather(t):
        q = idx_row(t, 0) * BATCH
        return [pltpu.async_copy(hs.at[sidx_v.at[pl.ds(q, SUP)]],
                                 rows[t % 2], gsem)]

    def fire_scatter(t):
        return [pltpu.async_copy(rows[t % 2].at[pl.ds(b * BATCH, BATCH)],
                                 accum.at[didx_v.at[idx_row(t, b)]], ssem,
                                 add=True) for b in range(KB)]

    load_pair(0)
    g_in = fire_gather(0)
    s_in = None
    for t in range(SUPERS):
        for d in g_in:
            d.wait()
        if s_in is not None:
            for d in s_in:
                d.wait()
        if t + 1 < SUPERS:
            if t % 2 == 1:
                load_pair((t + 1) // 2)
            g_in = fire_gather(t + 1)
        s_in = fire_scatter(t)
    for d in s_in:
        d.wait()

    plsc.subcore_barrier()
    pltpu.sync_copy(accum.at[pl.ds(s * ROWS_PT, ROWS_PT)],
                    out_p.at[c, pl.ds(s * ROWS_PT, ROWS_PT)])


def _sc_agg(hs_pad, src1d, dst2d):
    return pl.kernel(
        _sc_agg_body,
        out_type=jax.ShapeDtypeStruct((NC, NPAD, C), jnp.float32),
        mesh=_sc_mesh(),
        compiler_params=pltpu.CompilerParams(use_tc_tiling_on_sc=False),
        scratch_types=[
            pltpu.VMEM((4 * KB * BATCH,), jnp.int32),
            pltpu.VMEM((4 * KB, BATCH), jnp.int32),
            pltpu.VMEM((KB * BATCH, C), jnp.float32),
            pltpu.VMEM((KB * BATCH, C), jnp.float32),
            pltpu.VMEM_SHARED((NPAD, C), jnp.float32),
            pltpu.SemaphoreType.DMA,
            pltpu.SemaphoreType.DMA,
        ],
    )(hs_pad, src1d, dst2d)


def _sc_scat_body(y, op, rank3d, sy, sop, ridx_v, rows_v, rows2_v, sem):
    c = lax.axis_index("c")
    s = lax.axis_index("s")
    w = c * NS + s

    def do_batch(b):
        pltpu.sync_copy(rank3d.at[b, 0], ridx_v)
        pltpu.sync_copy(y.at[pl.ds(b * BATCH, BATCH)], rows_v)
        pltpu.sync_copy(op.at[pl.ds(b * BATCH, BATCH)], rows2_v)
        pltpu.async_copy(rows_v, sy.at[ridx_v], sem).wait()
        pltpu.async_copy(rows2_v, sop.at[ridx_v], sem).wait()

    do_batch(w * 2)
    do_batch(w * 2 + 1)

    @pl.when(w < NPAD // BATCH - 2 * NW)
    def _():
        do_batch(2 * NW + w)


def _sc_scat(y_pad, op_pad, rank3d):
    return pl.kernel(
        _sc_scat_body,
        out_type=(jax.ShapeDtypeStruct((NPAD, C), jnp.float32),
                  jax.ShapeDtypeStruct((NPAD, C), jnp.float32)),
        mesh=_sc_mesh(),
        compiler_params=pltpu.CompilerParams(use_tc_tiling_on_sc=False),
        scratch_types=[
            pltpu.VMEM((BATCH,), jnp.int32),
            pltpu.VMEM((BATCH, C), jnp.float32),
            pltpu.VMEM((BATCH, C), jnp.float32),
            pltpu.SemaphoreType.DMA,
        ],
    )(y_pad, op_pad, rank3d)


def _sc_gath_body(z, rank3d, out, ridx_v, rows_v, sem):
    c = lax.axis_index("c")
    s = lax.axis_index("s")
    w = c * NS + s

    def do_batch(b):
        pltpu.sync_copy(rank3d.at[b, 0], ridx_v)
        pltpu.async_copy(z.at[ridx_v], rows_v, sem).wait()
        pltpu.sync_copy(rows_v, out.at[pl.ds(b * BATCH, BATCH)])

    do_batch(w * 2)
    do_batch(w * 2 + 1)

    @pl.when(w < NPAD // BATCH - 2 * NW)
    def _():
        do_batch(2 * NW + w)


def _sc_gath(z, rank3d):
    return pl.kernel(
        _sc_gath_body,
        out_type=jax.ShapeDtypeStruct((NPAD, C), jnp.float32),
        mesh=_sc_mesh(),
        compiler_params=pltpu.CompilerParams(use_tc_tiling_on_sc=False),
        scratch_types=[
            pltpu.VMEM((BATCH,), jnp.int32),
            pltpu.VMEM((BATCH, C), jnp.float32),
            pltpu.SemaphoreType.DMA,
        ],
    )(z, rank3d)


def _tc1_body(x_ref, w1_ref, degt_ref, hs_ref, dis_ref):
    deg = degt_ref[:, 0:1] + degt_ref[:, 1:2] + 1.0
    dis = lax.rsqrt(deg)
    xw = jnp.dot(x_ref[...], w1_ref[...], preferred_element_type=jnp.float32)
    hs_ref[0:N, :] = xw * dis
    hs_ref[N:NPAD, :] = jnp.zeros((NPAD - N, HID), jnp.float32)
    dis_ref[...] = dis


def _tc1(x, W1, degt):
    return pl.pallas_call(
        _tc1_body,
        out_shape=(jax.ShapeDtypeStruct((NPAD, HID), jnp.float32),
                   jax.ShapeDtypeStruct((N, 1), jnp.float32)),
    )(x, W1, degt)


def _tc3_body(p_ref, hs1_ref, dis_ref, b1_ref, w2_ref, hs2_ref):
    agg = p_ref[0, 0:N, :] + p_ref[1, 0:N, :] - hs1_ref[0:N, :]
    h = jnp.maximum(agg * dis_ref[...] + b1_ref[...], 0.0)
    hw = jnp.dot(h, w2_ref[...], preferred_element_type=jnp.float32)
    hs2_ref[0:N, :] = hw * dis_ref[...]
    hs2_ref[N:NPAD, :] = jnp.zeros((NPAD - N, C), jnp.float32)


def _tc3(p, hs1, dis, b1, W2):
    return pl.pallas_call(
        _tc3_body,
        out_shape=jax.ShapeDtypeStruct((NPAD, C), jnp.float32),
    )(p, hs1, dis, b1, W2)


def _tc5_body(p_ref, hs2_ref, dis_ref, b2_ref, wp_ref, bp_ref, wlt_ref,
              bl_ref, y_ref, g_ref, op_ref):
    agg = p_ref[0, 0:N, :] + p_ref[1, 0:N, :] - hs2_ref[0:N, :]
    x1 = agg * dis_ref[...] + b2_ref[...]
    g = jnp.dot(x1, wp_ref[...], preferred_element_type=jnp.float32) + bp_ref[...]
    y_ref[0:N, :] = g * x1
    y_ref[N:NPAD, :] = jnp.zeros((NPAD - N, C), jnp.float32)
    g_ref[0:N, :] = g
    g_ref[N:NPAD, :] = jnp.full((NPAD - N, 1), jnp.inf, jnp.float32)
    op_ref[0:N, :] = (jnp.dot(x1, wlt_ref[...], preferred_element_type=jnp.float32)
                      + bl_ref[...])
    op_ref[N:NPAD, :] = jnp.zeros((NPAD - N, C), jnp.float32)


def _tc5(p, hs2, dis, b2, Wp, bp, Wl_top, bl):
    return pl.pallas_call(
        _tc5_body,
        out_shape=(jax.ShapeDtypeStruct((NPAD, C), jnp.float32),
                   jax.ShapeDtypeStruct((NPAD, 1), jnp.float32),
                   jax.ShapeDtypeStruct((NPAD, C), jnp.float32)),
    )(p, hs2, dis, b2, Wp, bp, Wl_top, bl)


def _key(v):
    b = lax.bitcast_convert_type(v, jnp.uint32)
    u = b ^ jnp.where(b >= jnp.uint32(0x80000000), jnp.uint32(0xFFFFFFFF),
                      jnp.uint32(0x80000000))
    return (u ^ jnp.uint32(0x80000000)).astype(jnp.int32)


RANK_BI = 256
RANK_BJ = 2048


def _rank_body(gcol_ref, grow_ref, rank_ref):
    i0 = pl.program_id(0) * RANK_BI
    ki = _key(gcol_ref[...])
    ii = lax.broadcasted_iota(jnp.int32, (RANK_BI, 1), 0) + i0
    acc = jnp.zeros((RANK_BI, 1), jnp.int32)
    for cj in range(NPAD // RANK_BJ):
        kj = _key(grow_ref[0:1, cj * RANK_BJ:(cj + 1) * RANK_BJ])
        jj = lax.broadcasted_iota(jnp.int32, (1, RANK_BJ), 1) + cj * RANK_BJ
        p = (kj < ki) | ((kj == ki) & (jj < ii))
        acc = acc + jnp.sum(p.astype(jnp.int32), axis=1, keepdims=True)
    rank_ref[...] = acc


def _rank(gcol, grow):
    return pl.pallas_call(
        _rank_body,
        grid=(NPAD // RANK_BI,),
        in_specs=[
            pl.BlockSpec((RANK_BI, 1), lambda i: (i, 0)),
            pl.BlockSpec((1, NPAD), lambda i: (0, 0)),
        ],
        out_specs=pl.BlockSpec((RANK_BI, 1), lambda i: (i, 0)),
        out_shape=jax.ShapeDtypeStruct((NPAD, 1), jnp.int32),
    )(gcol, grow)


def _conv7_body(sy_ref, sop_ref, w1s_ref, b1c_ref, w2s_ref, b2c_ref,
                wlb_ref, z_ref):
    def conv(xin, ws_ref, brow):
        out = brow
        for k in range(K):
            sh = k - K // 2
            if sh < 0:
                shifted = jnp.concatenate(
                    [jnp.zeros((-sh, C), jnp.float32), xin[: NPAD + sh]], axis=0)
            elif sh > 0:
                shifted = jnp.concatenate(
                    [xin[sh:], jnp.zeros((sh, C), jnp.float32)], axis=0)
            else:
                shifted = xin
            wk = ws_ref[pl.ds(k * C, C), :]
            out = out + jnp.dot(shifted, wk, preferred_element_type=jnp.float32)
        return out

    sy = sy_ref[...]
    s1 = jnp.maximum(conv(sy, w1s_ref, b1c_ref[...]), 0.0)
    rowmask = (lax.broadcasted_iota(jnp.int32, (NPAD, 1), 0) < N)
    s1 = jnp.where(rowmask, s1, 0.0)
    s2 = conv(s1, w2s_ref, b2c_ref[...])
    z_ref[...] = (jnp.dot(s2, wlb_ref[...], preferred_element_type=jnp.float32)
                  + sop_ref[...])


def _conv7(sy, sop, w1s, b1c, w2s, b2c, Wl_bot):
    return pl.pallas_call(
        _conv7_body,
        out_shape=jax.ShapeDtypeStruct((NPAD, C), jnp.float32),
    )(sy, sop, w1s, b1c, w2s, b2c, Wl_bot)


def kernel(x, W1, b1, W2, b2, Wp, bp, c1w, c1b, c2w, c2b, Wl, bl, edge_index):
    src, dst = edge_index[0], edge_index[1]
    pad = EPAD - E
    src_pad = jnp.concatenate([src, jnp.zeros((pad,), jnp.int32)])
    dst_pad = jnp.concatenate([dst, jnp.full((pad,), N, jnp.int32)])
    src2d = src_pad.reshape(EPAD // BATCH, BATCH)
    dst2d = dst_pad.reshape(EPAD // BATCH, BATCH)

    deg_p = _sc_deg(dst2d)
    degt = jnp.concatenate([deg_p[0, 0, :N].reshape(N, 1),
                            deg_p[1, 0, :N].reshape(N, 1)], axis=1)

    hs1, dis = _tc1(x, W1, degt)
    p1 = _sc_agg(hs1, src_pad, dst2d)

    hs2 = _tc3(p1, hs1, dis, b1.reshape(1, HID), W2)
    p2 = _sc_agg(hs2, src_pad, dst2d)

    Wl_top, Wl_bot = Wl[:C], Wl[C:]
    y, g, op = _tc5(p2, hs2, dis, b2.reshape(1, C), Wp, bp.reshape(1, 1),
                    Wl_top, bl.reshape(1, C))

    grow = g.reshape(1, NPAD)
    rank_col = _rank(g, grow)
    rank3d = rank_col.reshape(NPAD // BATCH, 1, BATCH)

    sy, sop = _sc_scat(y, op, rank3d)

    w1s = jnp.concatenate([c1w[:, :, k].T for k in range(K)], axis=0)
    w2s = jnp.concatenate([c2w[:, :, k].T for k in range(K)], axis=0)
    z = _conv7(sy, sop, w1s, c1b.reshape(1, C), w2s, c2b.reshape(1, C), Wl_bot)

    out_pad = _sc_gath(z, rank3d)
    return out_pad[:N]
